# baseline (speedup 1.0000x reference)
import functools

import jax
import jax.numpy as jnp
from jax import lax
from jax.experimental import pallas as pl
from jax.experimental.pallas import tpu as pltpu
from jax.experimental.pallas import tpu_sc as plsc

VOCAB = 1000000
D = 64
B = 16384
LQ = 20
LS = 50

_info = plsc.get_sparse_core_info()
NC = _info.num_cores
NS = _info.num_subcores
NW = NC * NS
BAGS_PER_W = B // NW
CB = 2
NCHUNK = BAGS_PER_W // CB


NBUF = 4


def _embed_bags(q_ids3, s_ids3, table):
    mesh = plsc.VectorSubcoreMesh(core_axis_name="c", subcore_axis_name="s")

    @functools.partial(
        pl.kernel,
        mesh=mesh,
        out_type=jax.ShapeDtypeStruct((B, 2 * D), jnp.float32),
        scratch_types=[
            pltpu.VMEM((NCHUNK, CB * LQ), jnp.int32),
            pltpu.VMEM((NCHUNK, CB * LS), jnp.int32),
            pltpu.VMEM((NBUF, CB * LQ, D), jnp.float32),
            pltpu.VMEM((NBUF, CB * LS, D), jnp.float32),
            pltpu.VMEM((NBUF, CB, 2 * D), jnp.float32),
            pltpu.SemaphoreType.DMA,
            pltpu.SemaphoreType.DMA,
            pltpu.SemaphoreType.DMA,
            pltpu.SemaphoreType.DMA,
        ],
        compiler_params=pltpu.CompilerParams(use_tc_tiling_on_sc=False),
    )
    def sc_kernel(q_hbm, s_hbm, table_hbm, out_hbm,
                  qidx_v, sidx_v, qrows_v, srows_v, h_v, s0, s1, s2, s3):
        sems = [s0, s1, s2, s3]
        wid = lax.axis_index("s") * NC + lax.axis_index("c")
        out_base = wid * BAGS_PER_W
        pltpu.sync_copy(q_hbm.at[wid], qidx_v)
        pltpu.sync_copy(s_hbm.at[wid], sidx_v)

        for b in range(NBUF):
            pltpu.async_copy(table_hbm.at[qidx_v.at[b]], qrows_v.at[b], sems[b])
            pltpu.async_copy(table_hbm.at[sidx_v.at[b]], srows_v.at[b], sems[b])

        def outer(i, _):
            for b in range(NBUF):
                j = i * NBUF + b
                qrows_b = qrows_v.at[b]
                srows_b = srows_v.at[b]
                pltpu.make_async_copy(table_hbm.at[qidx_v.at[j]], qrows_b, sems[b]).wait()
                pltpu.make_async_copy(table_hbm.at[sidx_v.at[j]], srows_b, sems[b]).wait()

                @pl.when(j >= NBUF)
                def _wait_h():
                    pltpu.make_async_copy(
                        h_v.at[b], out_hbm.at[pl.ds(0, CB)], sems[b]).wait()

                accq = [None] * (CB * (D // 16))
                for r in range(LQ):
                    for bag in range(CB):
                        for c in range(D // 16):
                            v = qrows_b[bag * LQ + r, pl.ds(c * 16, 16)]
                            k = bag * (D // 16) + c
                            accq[k] = v if r == 0 else accq[k] + v
                for bag in range(CB):
                    for c in range(D // 16):
                        h_v[b, bag, pl.ds(c * 16, 16)] = (
                            accq[bag * (D // 16) + c] * (1.0 / LQ))
                accs = [None] * (CB * (D // 16))
                for r in range(LS):
                    for bag in range(CB):
                        for c in range(D // 16):
                            v = srows_b[bag * LS + r, pl.ds(c * 16, 16)]
                            k = bag * (D // 16) + c
                            accs[k] = v if r == 0 else accs[k] + v
                for bag in range(CB):
                    for c in range(D // 16):
                        h_v[b, bag, pl.ds(D + c * 16, 16)] = (
                            accs[bag * (D // 16) + c] * (1.0 / LS))

                pltpu.async_copy(
                    h_v.at[b], out_hbm.at[pl.ds(out_base + j * CB, CB)], sems[b])

                @pl.when(j + NBUF < NCHUNK)
                def _refill():
                    pltpu.async_copy(
                        table_hbm.at[qidx_v.at[j + NBUF]], qrows_b, sems[b])
                    pltpu.async_copy(
                        table_hbm.at[sidx_v.at[j + NBUF]], srows_b, sems[b])
            return 0

        lax.fori_loop(0, NCHUNK // NBUF, outer, 0)
        for b in range(NBUF):
            pltpu.make_async_copy(
                h_v.at[b], out_hbm.at[pl.ds(0, CB)], sems[b]).wait()

    return sc_kernel(q_ids3, s_ids3, table)


def _mlp_body(h_ref, W1_ref, b1_ref, W2_ref, b2_ref, W3_ref, b3_ref, out_ref):
    h = h_ref[...]
    z1 = jnp.maximum(
        jnp.dot(h, W1_ref[...].T, preferred_element_type=jnp.float32) + b1_ref[...], 0.0)
    z2 = jnp.maximum(
        jnp.dot(z1, W2_ref[...].T, preferred_element_type=jnp.float32) + b2_ref[...], 0.0)
    out_ref[...] = jnp.sum(z2 * W3_ref[...], axis=1, keepdims=True) + b3_ref[0]


def _mlp(h, W1, b1, W2, b2, W3, b3):
    BLK = 1024
    grid = (B // BLK,)
    return pl.pallas_call(
        _mlp_body,
        grid=grid,
        in_specs=[
            pl.BlockSpec((BLK, 2 * D), lambda i: (i, 0)),
            pl.BlockSpec((128, 2 * D), lambda i: (0, 0)),
            pl.BlockSpec((128,), lambda i: (0,)),
            pl.BlockSpec((32, 128), lambda i: (0, 0)),
            pl.BlockSpec((32,), lambda i: (0,)),
            pl.BlockSpec((1, 32), lambda i: (0, 0)),
            pl.BlockSpec((1,), lambda i: (0,)),
        ],
        out_specs=pl.BlockSpec((BLK, 1), lambda i: (i, 0)),
        out_shape=jax.ShapeDtypeStruct((B, 1), jnp.float32),
        compiler_params=pltpu.CompilerParams(
            dimension_semantics=("parallel",),
        ),
    )(h, W1, b1, W2, b2, W3, b3)


def kernel(q_ids, s_ids, table, W1, b1, W2, b2, W3, b3):
    q3 = q_ids.reshape(NW, NCHUNK, CB * LQ).astype(jnp.int32)
    s3 = s_ids.reshape(NW, NCHUNK, CB * LS).astype(jnp.int32)
    h = _embed_bags(q3, s3, table)
    out = _mlp(h, W1, b1, W2, b2, W3, b3)
    return out.squeeze(-1)


# speedup vs baseline: 1.4246x; 1.4246x over previous
import functools

import jax
import jax.numpy as jnp
from jax import lax
from jax.experimental import pallas as pl
from jax.experimental.pallas import tpu as pltpu
from jax.experimental.pallas import tpu_sc as plsc

VOCAB = 1000000
D = 64
B = 16384
LQ = 20
LS = 50

_info = plsc.get_sparse_core_info()
NC = _info.num_cores
NS = _info.num_subcores
NW = NC * NS
BAGS_PER_W = B // NW
CB = 4
NCHUNK = BAGS_PER_W // CB
S_SPLIT = (104, 96)


NBUF = 4


def _embed_bags(q_ids3, s_ids3, table):
    mesh = plsc.VectorSubcoreMesh(core_axis_name="c", subcore_axis_name="s")

    @functools.partial(
        pl.kernel,
        mesh=mesh,
        out_type=jax.ShapeDtypeStruct((B, 2 * D), jnp.float32),
        scratch_types=[
            pltpu.VMEM((BAGS_PER_W * LQ,), jnp.int32),
            pltpu.VMEM((BAGS_PER_W * LS,), jnp.int32),
            pltpu.VMEM((NBUF, CB * LQ, D), jnp.float32),
            pltpu.VMEM((NBUF, CB * LS, D), jnp.float32),
            pltpu.VMEM((NBUF, CB, 2 * D), jnp.float32),
            pltpu.SemaphoreType.DMA,
            pltpu.SemaphoreType.DMA,
            pltpu.SemaphoreType.DMA,
            pltpu.SemaphoreType.DMA,
        ],
        compiler_params=pltpu.CompilerParams(use_tc_tiling_on_sc=False),
    )
    def sc_kernel(q_hbm, s_hbm, table_hbm, out_hbm,
                  qidx_v, sidx_v, qrows_v, srows_v, h_v, s0, s1, s2, s3):
        sems = [s0, s1, s2, s3]
        wid = lax.axis_index("s") * NC + lax.axis_index("c")
        out_base = wid * BAGS_PER_W
        pltpu.sync_copy(
            q_hbm.at[pl.ds(wid * BAGS_PER_W * LQ, BAGS_PER_W * LQ)], qidx_v)
        pltpu.sync_copy(
            s_hbm.at[pl.ds(wid * BAGS_PER_W * LS, BAGS_PER_W * LS)], sidx_v)

        def gathers(j, b):
            s0_len, s1_len = S_SPLIT
            return [
                (table_hbm.at[qidx_v.at[pl.ds(j * CB * LQ, CB * LQ)]],
                 qrows_v.at[b]),
                (table_hbm.at[sidx_v.at[pl.ds(j * CB * LS, s0_len)]],
                 srows_v.at[b, pl.ds(0, s0_len)]),
                (table_hbm.at[sidx_v.at[pl.ds(j * CB * LS + s0_len, s1_len)]],
                 srows_v.at[b, pl.ds(s0_len, s1_len)]),
            ]

        for b in range(NBUF):
            for src, dst in gathers(b, b):
                pltpu.async_copy(src, dst, sems[b])

        def outer(i, _):
            for b in range(NBUF):
                j = i * NBUF + b
                qrows_b = qrows_v.at[b]
                srows_b = srows_v.at[b]
                for src, dst in gathers(j, b):
                    pltpu.make_async_copy(src, dst, sems[b]).wait()

                @pl.when(j >= NBUF)
                def _wait_h():
                    pltpu.make_async_copy(
                        h_v.at[b], out_hbm.at[pl.ds(0, CB)], sems[b]).wait()

                def bag_body(bag, _):
                    acc = [qrows_b[bag * LQ, pl.ds(c * 16, 16)]
                           for c in range(D // 16)]
                    for r in range(1, LQ):
                        for c in range(D // 16):
                            acc[c] = acc[c] + qrows_b[bag * LQ + r, pl.ds(c * 16, 16)]
                    for c in range(D // 16):
                        h_v[b, bag, pl.ds(c * 16, 16)] = acc[c] * (1.0 / LQ)
                    acc = [srows_b[bag * LS, pl.ds(c * 16, 16)]
                           for c in range(D // 16)]
                    for r in range(1, LS):
                        for c in range(D // 16):
                            acc[c] = acc[c] + srows_b[bag * LS + r, pl.ds(c * 16, 16)]
                    for c in range(D // 16):
                        h_v[b, bag, pl.ds(D + c * 16, 16)] = acc[c] * (1.0 / LS)
                    return 0

                lax.fori_loop(0, CB, bag_body, 0)

                pltpu.async_copy(
                    h_v.at[b], out_hbm.at[pl.ds(out_base + j * CB, CB)], sems[b])

                @pl.when(j + NBUF < NCHUNK)
                def _refill():
                    for src, dst in gathers(j + NBUF, b):
                        pltpu.async_copy(src, dst, sems[b])
            return 0

        lax.fori_loop(0, NCHUNK // NBUF, outer, 0)
        for b in range(NBUF):
            pltpu.make_async_copy(
                h_v.at[b], out_hbm.at[pl.ds(0, CB)], sems[b]).wait()

    return sc_kernel(q_ids3, s_ids3, table)


def _mlp_body(h_ref, W1_ref, b1_ref, W2_ref, b2_ref, W3_ref, b3_ref, out_ref):
    h = h_ref[...]
    z1 = jnp.maximum(
        jnp.dot(h, W1_ref[...].T, preferred_element_type=jnp.float32) + b1_ref[...], 0.0)
    z2 = jnp.maximum(
        jnp.dot(z1, W2_ref[...].T, preferred_element_type=jnp.float32) + b2_ref[...], 0.0)
    out_ref[...] = jnp.sum(z2 * W3_ref[...], axis=1, keepdims=True) + b3_ref[0]


def _mlp(h, W1, b1, W2, b2, W3, b3):
    BLK = 1024
    grid = (B // BLK,)
    return pl.pallas_call(
        _mlp_body,
        grid=grid,
        in_specs=[
            pl.BlockSpec((BLK, 2 * D), lambda i: (i, 0)),
            pl.BlockSpec((128, 2 * D), lambda i: (0, 0)),
            pl.BlockSpec((128,), lambda i: (0,)),
            pl.BlockSpec((32, 128), lambda i: (0, 0)),
            pl.BlockSpec((32,), lambda i: (0,)),
            pl.BlockSpec((1, 32), lambda i: (0, 0)),
            pl.BlockSpec((1,), lambda i: (0,)),
        ],
        out_specs=pl.BlockSpec((BLK, 1), lambda i: (i, 0)),
        out_shape=jax.ShapeDtypeStruct((B, 1), jnp.float32),
        compiler_params=pltpu.CompilerParams(
            dimension_semantics=("parallel",),
        ),
    )(h, W1, b1, W2, b2, W3, b3)


def kernel(q_ids, s_ids, table, W1, b1, W2, b2, W3, b3):
    q_flat = q_ids.reshape(B * LQ).astype(jnp.int32)
    s_flat = s_ids.reshape(B * LS).astype(jnp.int32)
    h = _embed_bags(q_flat, s_flat, table)
    out = _mlp(h, W1, b1, W2, b2, W3, b3)
    return out.squeeze(-1)
